# baseline (speedup 1.0000x reference)
import functools

import jax
import jax.numpy as jnp
from jax import lax
from jax.experimental import pallas as pl
from jax.experimental.pallas import tpu as pltpu
from jax.experimental.pallas import tpu_sc as plsc

_CHUNK = 128
_GRP = 4
_GROWS = _GRP * _CHUNK
_NW = 32


def _build_gather(n_chunks, d_model):
    mesh = plsc.VectorSubcoreMesh(core_axis_name="c", subcore_axis_name="s")
    n_rows = n_chunks * _CHUNK
    n_groups = n_chunks // _GRP
    num_cores = 2

    @functools.partial(
        pl.kernel,
        mesh=mesh,
        compiler_params=pltpu.CompilerParams(use_tc_tiling_on_sc=False),
        out_type=jax.ShapeDtypeStruct((_NW * n_rows, d_model), jnp.float32),
        scratch_types=[
            pltpu.VMEM((n_chunks, _CHUNK), jnp.int32),
            pltpu.VMEM((2, _GROWS, d_model), jnp.float32),
            pltpu.SemaphoreType.DMA((2,)),
            pltpu.SemaphoreType.DMA((2,)),
        ],
    )
    def gather_kernel(table_hbm, idx_hbm, out_hbm, idx_v, rows_v, gsem, ssem):
        wid = lax.axis_index("s") * num_cores + lax.axis_index("c")
        base = wid * n_rows
        pltpu.sync_copy(idx_hbm.at[wid], idx_v)

        def issue_gathers(g, par):
            for q in range(_GRP):
                pltpu.async_copy(
                    table_hbm.at[idx_v.at[g * _GRP + q]],
                    rows_v.at[par, pl.ds(q * _CHUNK, _CHUNK)],
                    gsem.at[par],
                )

        def wait_gathers(par):
            pltpu.make_async_copy(
                table_hbm.at[pl.ds(0, _GROWS)], rows_v.at[par], gsem.at[par]
            ).wait()

        def start_store(g, par):
            pltpu.async_copy(
                rows_v.at[par],
                out_hbm.at[pl.ds(base + g * _GROWS, _GROWS)],
                ssem.at[par],
            )

        def wait_store(par):
            pltpu.make_async_copy(
                rows_v.at[par], out_hbm.at[pl.ds(base, _GROWS)], ssem.at[par]
            ).wait()

        issue_gathers(0, 0)
        issue_gathers(1, 1)
        wait_gathers(0)
        start_store(0, 0)

        def body(k, carry):
            g = 2 * k + 1
            wait_store(0)
            issue_gathers(g + 1, 0)
            wait_gathers(1)
            start_store(g, 1)
            wait_store(1)
            issue_gathers(g + 2, 1)
            wait_gathers(0)
            start_store(g + 1, 0)
            return carry

        lax.fori_loop(0, (n_groups - 2) // 2, body, 0)

        wait_gathers(1)
        start_store(n_groups - 1, 1)
        wait_store(0)
        wait_store(1)

    return gather_kernel


def _build_reemit(n128):
    mesh = plsc.VectorSubcoreMesh(core_axis_name="c", subcore_axis_name="s")
    num_cores = 2
    per = n128 // _NW
    half = per // 2

    @functools.partial(
        pl.kernel,
        mesh=mesh,
        compiler_params=pltpu.CompilerParams(use_tc_tiling_on_sc=False),
        out_type=jax.ShapeDtypeStruct((n128, 128), jnp.float32),
        scratch_types=[pltpu.SemaphoreType.DMA((2,))],
    )
    def reemit_kernel(in_hbm, out_hbm, sem):
        wid = lax.axis_index("s") * num_cores + lax.axis_index("c")
        base = wid * per
        for c in range(2):
            pltpu.async_copy(
                in_hbm.at[pl.ds(base + c * half, half)],
                out_hbm.at[pl.ds(base + c * half, half)],
                sem.at[c],
            )
        for c in range(2):
            pltpu.make_async_copy(
                in_hbm.at[pl.ds(base, half)],
                out_hbm.at[pl.ds(base, half)],
                sem.at[c],
            ).wait()

    return reemit_kernel


def kernel(token_ids, weight):
    b, s = token_ids.shape
    d_model = weight.shape[1]
    total = b * s
    flat = token_ids.reshape(-1).astype(jnp.int32)

    grain = _NW * _GROWS * 2
    padded = -(-total // grain) * grain
    if padded != total:
        flat = jnp.concatenate(
            [flat, jnp.zeros((padded - total,), jnp.int32)], axis=0
        )
    per_worker = padded // _NW
    idx3 = flat.reshape(_NW, per_worker // _CHUNK, _CHUNK)

    vocab = weight.shape[0]
    w128 = weight.reshape(vocab * d_model // 128, 128)
    w128 = lax.optimization_barrier(w128)
    w64 = w128.reshape(vocab, d_model)

    gather = _build_gather(per_worker // _CHUNK, d_model)
    mid = gather(w64, idx3)

    n128 = padded * d_model // 128
    reemit = _build_reemit(n128)
    out128 = reemit(mid.reshape(n128, 128))
    return out128.reshape(padded, d_model)[:total].reshape(b, s, d_model)


# speedup vs baseline: 5.4110x; 5.4110x over previous
import functools

import jax
import jax.numpy as jnp
from jax import lax
from jax.experimental import pallas as pl
from jax.experimental.pallas import tpu as pltpu
from jax.experimental.pallas import tpu_sc as plsc

_CHUNK = 128
_GRP = 4
_GROWS = _GRP * _CHUNK
_NW = 32


def _build_gather(n_chunks, d_model):
    mesh = plsc.VectorSubcoreMesh(core_axis_name="c", subcore_axis_name="s")
    n_rows = n_chunks * _CHUNK
    n_groups = n_chunks // _GRP
    num_cores = 2

    @functools.partial(
        pl.kernel,
        mesh=mesh,
        compiler_params=pltpu.CompilerParams(use_tc_tiling_on_sc=False),
        out_type=jax.ShapeDtypeStruct((_NW * n_rows, d_model), jnp.float32),
        scratch_types=[
            pltpu.VMEM((n_chunks, _CHUNK), jnp.int32),
            pltpu.VMEM((2, _GROWS, d_model), jnp.float32),
            pltpu.SemaphoreType.DMA((2,)),
            pltpu.SemaphoreType.DMA((2,)),
        ],
    )
    def gather_kernel(table_hbm, idx_hbm, out_hbm, idx_v, rows_v, gsem, ssem):
        wid = lax.axis_index("s") * num_cores + lax.axis_index("c")
        base = wid * n_rows
        pltpu.sync_copy(idx_hbm.at[wid], idx_v)

        def issue_gathers(g, par):
            for q in range(_GRP):
                pltpu.async_copy(
                    table_hbm.at[idx_v.at[g * _GRP + q]],
                    rows_v.at[par, pl.ds(q * _CHUNK, _CHUNK)],
                    gsem.at[par],
                )

        def wait_gathers(par):
            pltpu.make_async_copy(
                table_hbm.at[pl.ds(0, _GROWS)], rows_v.at[par], gsem.at[par]
            ).wait()

        def start_store(g, par):
            pltpu.async_copy(
                rows_v.at[par],
                out_hbm.at[pl.ds(base + g * _GROWS, _GROWS)],
                ssem.at[par],
            )

        def wait_store(par):
            pltpu.make_async_copy(
                rows_v.at[par], out_hbm.at[pl.ds(base, _GROWS)], ssem.at[par]
            ).wait()

        issue_gathers(0, 0)
        issue_gathers(1, 1)
        wait_gathers(0)
        start_store(0, 0)

        def body(k, carry):
            g = 2 * k + 1
            wait_store(0)
            issue_gathers(g + 1, 0)
            wait_gathers(1)
            start_store(g, 1)
            wait_store(1)
            issue_gathers(g + 2, 1)
            wait_gathers(0)
            start_store(g + 1, 0)
            return carry

        lax.fori_loop(0, (n_groups - 2) // 2, body, 0)

        wait_gathers(1)
        start_store(n_groups - 1, 1)
        wait_store(0)
        wait_store(1)

    return gather_kernel


_RCH = 256


def _build_reemit(n128):
    mesh = plsc.VectorSubcoreMesh(core_axis_name="c", subcore_axis_name="s")
    num_cores = 2
    per = n128 // _NW
    n_chunks = per // _RCH

    @functools.partial(
        pl.kernel,
        mesh=mesh,
        compiler_params=pltpu.CompilerParams(use_tc_tiling_on_sc=False),
        out_type=jax.ShapeDtypeStruct((n128, 128), jnp.float32),
        scratch_types=[
            pltpu.VMEM((2, _RCH, 128), jnp.float32),
            pltpu.SemaphoreType.DMA((2,)),
            pltpu.SemaphoreType.DMA((2,)),
        ],
    )
    def reemit_kernel(in_hbm, out_hbm, buf, isem, osem):
        wid = lax.axis_index("s") * num_cores + lax.axis_index("c")
        base = wid * per

        def load(c, par):
            pltpu.async_copy(
                in_hbm.at[pl.ds(base + c * _RCH, _RCH)], buf.at[par], isem.at[par]
            )

        def wait_load(par):
            pltpu.make_async_copy(
                in_hbm.at[pl.ds(base, _RCH)], buf.at[par], isem.at[par]
            ).wait()

        def store(c, par):
            pltpu.async_copy(
                buf.at[par], out_hbm.at[pl.ds(base + c * _RCH, _RCH)], osem.at[par]
            )

        def wait_store(par):
            pltpu.make_async_copy(
                buf.at[par], out_hbm.at[pl.ds(base, _RCH)], osem.at[par]
            ).wait()

        load(0, 0)
        load(1, 1)
        wait_load(0)
        store(0, 0)

        def body(k, carry):
            g = 2 * k + 1
            wait_store(0)
            load(g + 1, 0)
            wait_load(1)
            store(g, 1)
            wait_store(1)
            load(g + 2, 1)
            wait_load(0)
            store(g + 1, 0)
            return carry

        lax.fori_loop(0, (n_chunks - 2) // 2, body, 0)

        wait_load(1)
        store(n_chunks - 1, 1)
        wait_store(0)
        wait_store(1)

    return reemit_kernel


def kernel(token_ids, weight):
    b, s = token_ids.shape
    d_model = weight.shape[1]
    total = b * s
    flat = token_ids.reshape(-1).astype(jnp.int32)

    grain = _NW * _GROWS * 2
    padded = -(-total // grain) * grain
    if padded != total:
        flat = jnp.concatenate(
            [flat, jnp.zeros((padded - total,), jnp.int32)], axis=0
        )
    per_worker = padded // _NW
    idx3 = flat.reshape(_NW, per_worker // _CHUNK, _CHUNK)

    vocab = weight.shape[0]
    w128 = weight.reshape(vocab * d_model // 128, 128)
    w128 = lax.optimization_barrier(w128)
    w64 = w128.reshape(vocab, d_model)

    gather = _build_gather(per_worker // _CHUNK, d_model)
    mid = gather(w64, idx3)

    n128 = padded * d_model // 128
    reemit = _build_reemit(n128)
    out128 = reemit(mid.reshape(n128, 128))
    return out128.reshape(padded, d_model)[:total].reshape(b, s, d_model)


# speedup vs baseline: 6.0675x; 1.1213x over previous
import functools

import jax
import jax.numpy as jnp
from jax import lax
from jax.experimental import pallas as pl
from jax.experimental.pallas import tpu as pltpu
from jax.experimental.pallas import tpu_sc as plsc

_CHUNK = 128
_GRP = 4
_GROWS = _GRP * _CHUNK
_NW = 32


def _build_gather(n_chunks, d_model):
    mesh = plsc.VectorSubcoreMesh(core_axis_name="c", subcore_axis_name="s")
    n_rows = n_chunks * _CHUNK
    n_groups = n_chunks // _GRP
    num_cores = 2

    @functools.partial(
        pl.kernel,
        mesh=mesh,
        compiler_params=pltpu.CompilerParams(use_tc_tiling_on_sc=False),
        out_type=jax.ShapeDtypeStruct((_NW * n_rows, d_model), jnp.float32),
        scratch_types=[
            pltpu.VMEM((n_chunks, _CHUNK), jnp.int32),
            pltpu.VMEM((2, _GROWS, d_model), jnp.float32),
            pltpu.SemaphoreType.DMA((2,)),
            pltpu.SemaphoreType.DMA((2,)),
        ],
    )
    def gather_kernel(table_hbm, idx_hbm, out_hbm, idx_v, rows_v, gsem, ssem):
        wid = lax.axis_index("s") * num_cores + lax.axis_index("c")
        base = wid * n_rows
        pltpu.sync_copy(idx_hbm.at[wid], idx_v)

        def issue_gathers(g, par):
            for q in range(_GRP):
                pltpu.async_copy(
                    table_hbm.at[idx_v.at[g * _GRP + q]],
                    rows_v.at[par, pl.ds(q * _CHUNK, _CHUNK)],
                    gsem.at[par],
                )

        def wait_gathers(par):
            pltpu.make_async_copy(
                table_hbm.at[pl.ds(0, _GROWS)], rows_v.at[par], gsem.at[par]
            ).wait()

        def start_store(g, par):
            pltpu.async_copy(
                rows_v.at[par],
                out_hbm.at[pl.ds(base + g * _GROWS, _GROWS)],
                ssem.at[par],
            )

        def wait_store(par):
            pltpu.make_async_copy(
                rows_v.at[par], out_hbm.at[pl.ds(base, _GROWS)], ssem.at[par]
            ).wait()

        issue_gathers(0, 0)
        issue_gathers(1, 1)
        wait_gathers(0)
        start_store(0, 0)

        def body(k, carry):
            g = 2 * k + 1
            wait_store(0)
            issue_gathers(g + 1, 0)
            wait_gathers(1)
            start_store(g, 1)
            wait_store(1)
            issue_gathers(g + 2, 1)
            wait_gathers(0)
            start_store(g + 1, 0)
            return carry

        lax.fori_loop(0, (n_groups - 2) // 2, body, 0)

        wait_gathers(1)
        start_store(n_groups - 1, 1)
        wait_store(0)
        wait_store(1)

    return gather_kernel


def kernel(token_ids, weight):
    b, s = token_ids.shape
    d_model = weight.shape[1]
    total = b * s
    flat = token_ids.reshape(-1).astype(jnp.int32)

    grain = _NW * _GROWS * 2
    padded = -(-total // grain) * grain
    if padded != total:
        flat = jnp.concatenate(
            [flat, jnp.zeros((padded - total,), jnp.int32)], axis=0
        )
    per_worker = padded // _NW
    idx3 = flat.reshape(_NW, per_worker // _CHUNK, _CHUNK)

    gather = _build_gather(per_worker // _CHUNK, d_model)
    out = gather(weight, idx3)
    return out[:total].reshape(b, s, d_model)
